# baseline (speedup 1.0000x reference)
import jax
import jax.numpy as jnp
from jax.experimental import pallas as pl
from jax.experimental.pallas import tpu as pltpu

N_LEVELS = 8
K = 1024
D = 256
BLOCK_B = 4096
N_SPLIT = 2


def _rvq_kernel(x_ref, cbm2_ref, csplit_ref, cnorm_ref, out_ref):
    b = x_ref.shape[0]
    h = b // N_SPLIT
    lane_iota = jax.lax.broadcasted_iota(jnp.int32, (h, K), 1)

    def level_step(r, level, row0):
        m2p = jax.lax.dot_general(
            r, cbm2_ref[level], (((1,), (1,)), ((), ())),
            preferred_element_type=jnp.float32,
        )
        d2 = cnorm_ref[level][None, :] + m2p
        idx = jnp.argmin(d2, axis=1).astype(jnp.int32)
        out_ref[pl.ds(row0, h), level] = idx
        if level == N_LEVELS - 1:
            return r
        onehot = (lane_iota == idx[:, None]).astype(jnp.float32).astype(jnp.bfloat16)
        s = jax.lax.dot_general(
            onehot, csplit_ref[level], (((1,), (0,)), ((), ())),
            preferred_element_type=jnp.float32,
        )
        sel = (s[:, :D] + s[:, D:2 * D]) + s[:, 2 * D:]
        return r - sel

    rs = [x_ref[pl.ds(j * h, h), :] for j in range(N_SPLIT)]
    for level in range(N_LEVELS):
        rs = [level_step(rs[j], level, j * h) for j in range(N_SPLIT)]


@jax.jit
def kernel(x, codebooks):
    n = x.shape[0]
    cnorms = jnp.sum(codebooks * codebooks, axis=-1)
    cbm2 = -2.0 * codebooks
    mask = jnp.uint32(0xFFFF0000)
    bits = jax.lax.bitcast_convert_type(codebooks, jnp.uint32)
    b0 = jax.lax.bitcast_convert_type(bits & mask, jnp.float32)
    r1 = codebooks - b0
    b1 = jax.lax.bitcast_convert_type(
        jax.lax.bitcast_convert_type(r1, jnp.uint32) & mask, jnp.float32)
    b2 = r1 - b1
    csplit = jnp.concatenate(
        [b0.astype(jnp.bfloat16), b1.astype(jnp.bfloat16),
         b2.astype(jnp.bfloat16)], axis=-1)

    grid = (n // BLOCK_B,)
    out = pl.pallas_call(
        _rvq_kernel,
        grid=grid,
        in_specs=[
            pl.BlockSpec((BLOCK_B, D), lambda i: (i, 0)),
            pl.BlockSpec((N_LEVELS, K, D), lambda i: (0, 0, 0)),
            pl.BlockSpec((N_LEVELS, K, 3 * D), lambda i: (0, 0, 0)),
            pl.BlockSpec((N_LEVELS, K), lambda i: (0, 0)),
        ],
        out_specs=pl.BlockSpec((BLOCK_B, N_LEVELS), lambda i: (i, 0)),
        out_shape=jax.ShapeDtypeStruct((n, N_LEVELS), jnp.int32),
        compiler_params=pltpu.CompilerParams(
            dimension_semantics=("parallel",)),
    )(x, cbm2, csplit, cnorms)
    return out


# speedup vs baseline: 1.2114x; 1.2114x over previous
import jax
import jax.numpy as jnp
from jax.experimental import pallas as pl
from jax.experimental.pallas import tpu as pltpu

N_LEVELS = 8
K = 1024
D = 256
BLOCK_B = 2048
N_SPLIT = 2


def _rvq_kernel(x_ref, cbm2_ref, csplit_ref, cnorm_ref, out_ref):
    b = x_ref.shape[0]
    h = b // N_SPLIT
    lane_iota = jax.lax.broadcasted_iota(jnp.int32, (h, K), 1)

    def level_step(r, level, row0):
        m2p = jax.lax.dot_general(
            r, cbm2_ref[level], (((1,), (1,)), ((), ())),
            preferred_element_type=jnp.float32,
        )
        d2 = cnorm_ref[level][None, :] + m2p
        idx = jnp.argmin(d2, axis=1).astype(jnp.int32)
        out_ref[pl.ds(row0, h), level] = idx
        if level == N_LEVELS - 1:
            return r
        onehot = (lane_iota == idx[:, None]).astype(jnp.float32).astype(jnp.bfloat16)
        s = jax.lax.dot_general(
            onehot, csplit_ref[level], (((1,), (0,)), ((), ())),
            preferred_element_type=jnp.float32,
        )
        sel = (s[:, :D] + s[:, D:2 * D]) + s[:, 2 * D:]
        return r - sel

    rs = [x_ref[pl.ds(j * h, h), :] for j in range(N_SPLIT)]
    for level in range(N_LEVELS):
        rs = [level_step(rs[j], level, j * h) for j in range(N_SPLIT)]


@jax.jit
def kernel(x, codebooks):
    n = x.shape[0]
    cnorms = jnp.sum(codebooks * codebooks, axis=-1)
    cbm2 = -2.0 * codebooks
    mask = jnp.uint32(0xFFFF0000)
    bits = jax.lax.bitcast_convert_type(codebooks, jnp.uint32)
    b0 = jax.lax.bitcast_convert_type(bits & mask, jnp.float32)
    r1 = codebooks - b0
    b1 = jax.lax.bitcast_convert_type(
        jax.lax.bitcast_convert_type(r1, jnp.uint32) & mask, jnp.float32)
    b2 = r1 - b1
    csplit = jnp.concatenate(
        [b0.astype(jnp.bfloat16), b1.astype(jnp.bfloat16),
         b2.astype(jnp.bfloat16)], axis=-1)

    grid = (n // BLOCK_B,)
    out = pl.pallas_call(
        _rvq_kernel,
        grid=grid,
        in_specs=[
            pl.BlockSpec((BLOCK_B, D), lambda i: (i, 0)),
            pl.BlockSpec((N_LEVELS, K, D), lambda i: (0, 0, 0)),
            pl.BlockSpec((N_LEVELS, K, 3 * D), lambda i: (0, 0, 0)),
            pl.BlockSpec((N_LEVELS, K), lambda i: (0, 0)),
        ],
        out_specs=pl.BlockSpec((BLOCK_B, N_LEVELS), lambda i: (i, 0)),
        out_shape=jax.ShapeDtypeStruct((n, N_LEVELS), jnp.int32),
        compiler_params=pltpu.CompilerParams(
            dimension_semantics=("parallel",)),
    )(x, cbm2, csplit, cnorms)
    return out
